# Initial kernel scaffold; baseline (speedup 1.0000x reference)
import jax
import jax.numpy as jnp
from jax.experimental import pallas as pl


def kernel(block_outputs, pos_edge_index, neg_edge_index):
    raise NotImplementedError("write your pallas kernel here")


# speedup vs baseline: 5.1722x; 5.1722x over previous
import functools

import jax
import jax.numpy as jnp
from jax import lax
from jax.experimental import pallas as pl
from jax.experimental.pallas import tpu as pltpu
from jax.experimental.pallas import tpu_sc as plsc

NW = 32
C = 80
LANES = 16


def _sc_scores(table, idx_u, idx_v, n_chunks, per_worker, total_edges):
    D = table.shape[1]
    mesh = plsc.VectorSubcoreMesh(core_axis_name="c", subcore_axis_name="s")

    @functools.partial(
        pl.kernel,
        out_type=jax.ShapeDtypeStruct((total_edges,), jnp.float32),
        mesh=mesh,
        scratch_types=[
            pltpu.VMEM((n_chunks, C), jnp.int32),
            pltpu.VMEM((n_chunks, C), jnp.int32),
            pltpu.VMEM((C, D), jnp.float32),
            pltpu.VMEM((C, D), jnp.float32),
            pltpu.VMEM((C, D), jnp.float32),
            pltpu.VMEM((C, D), jnp.float32),
            pltpu.VMEM((per_worker,), jnp.float32),
            pltpu.SemaphoreType.DMA,
            pltpu.SemaphoreType.DMA,
            pltpu.SemaphoreType.DMA,
            pltpu.SemaphoreType.DMA,
        ],
    )
    def k(table_hbm, iu_hbm, iv_hbm, out_hbm,
          iu, iv, ub0, vb0, ub1, vb1, scores, su0, sv0, su1, sv1):
        wid = lax.axis_index("s") * 2 + lax.axis_index("c")
        bufs = ((ub0, vb0, su0, sv0), (ub1, vb1, su1, sv1))

        pltpu.sync_copy(iu_hbm.at[wid], iu)
        pltpu.sync_copy(iv_hbm.at[wid], iv)

        def fire(b, j):
            ub, vb, su, sv = bufs[b]
            pltpu.async_copy(table_hbm.at[iu.at[j]], ub, su)
            pltpu.async_copy(table_hbm.at[iv.at[j]], vb, sv)

        def wait(b, j):
            ub, vb, su, sv = bufs[b]
            pltpu.make_async_copy(table_hbm.at[iu.at[j]], ub, su).wait()
            pltpu.make_async_copy(table_hbm.at[iv.at[j]], vb, sv).wait()

        fire(0, 0)
        fire(1, 1)

        lane = lax.iota(jnp.int32, LANES)
        idx_even = (2 * lane) & (LANES - 1)
        idx_odd = (2 * lane + 1) & (LANES - 1)
        low_half = lane < (LANES // 2)

        def gtr(x, idx):
            return x.at[idx].get(mode="promise_in_bounds")

        def hadd(a, b):
            sa = gtr(a, idx_even) + gtr(a, idx_odd)
            sb = gtr(b, idx_even) + gtr(b, idx_odd)
            return jnp.where(low_half, sa, sb)

        def chunk_body(i, carry):
            j0 = 2 * i
            for b in range(2):
                j = j0 + b
                ub, vb = bufs[b][0], bufs[b][1]
                wait(b, j)

                def group_body(g, c2):
                    ps = []
                    for el in range(LANES):
                        e = g * LANES + el
                        acc = ub[e, pl.ds(0, LANES)] * vb[e, pl.ds(0, LANES)]
                        for d in range(1, D // LANES):
                            acc = acc + (ub[e, pl.ds(d * LANES, LANES)]
                                         * vb[e, pl.ds(d * LANES, LANES)])
                        ps.append(acc)
                    while len(ps) > 1:
                        ps = [hadd(ps[k], ps[k + 1]) for k in range(0, len(ps), 2)]
                    scores[pl.ds(j * C + g * LANES, LANES)] = ps[0]
                    return c2

                lax.fori_loop(0, C // LANES, group_body, 0)

                @pl.when(j + 2 < n_chunks)
                def _():
                    fire(b, j + 2)
            return carry

        lax.fori_loop(0, n_chunks // 2, chunk_body, 0)
        pltpu.sync_copy(scores, out_hbm.at[pl.ds(wid * per_worker, per_worker)])

    return k(table, idx_u, idx_v)


def _tc_loss(scores2d, n_edges):
    pos_rows = scores2d.shape[0] // 2

    def body(s_ref, out_ref):
        s = s_ref[...]
        row = lax.broadcasted_iota(jnp.int32, s.shape, 0)
        pos_m = row < pos_rows
        t = jnp.where(pos_m, s, -s)
        ls = jnp.minimum(t, 0.0) - jnp.log1p(jnp.exp(-jnp.abs(t)))
        pos_loss = -jnp.sum(jnp.where(pos_m, ls, 0.0)) / n_edges
        neg_loss = -jnp.sum(jnp.where(pos_m, 0.0, ls)) / n_edges
        out_ref[0] = pos_loss + neg_loss
        out_ref[1] = pos_loss
        out_ref[2] = neg_loss

    return pl.pallas_call(
        body,
        out_shape=jax.ShapeDtypeStruct((3,), jnp.float32),
        in_specs=[pl.BlockSpec(memory_space=pltpu.VMEM)],
        out_specs=pl.BlockSpec(memory_space=pltpu.SMEM),
    )(scores2d)


def kernel(block_outputs, pos_edge_index, neg_edge_index):
    E = pos_edge_index.shape[1]
    total = 2 * E
    per_worker = total // NW
    n_chunks = per_worker // C

    idx_u = jnp.concatenate(
        [pos_edge_index[0], neg_edge_index[0]]).reshape(NW, n_chunks, C)
    idx_v = jnp.concatenate(
        [pos_edge_index[1], neg_edge_index[1]]).reshape(NW, n_chunks, C)

    scores = _sc_scores(block_outputs, idx_u, idx_v, n_chunks, per_worker, total)
    out3 = _tc_loss(scores.reshape(-1, 128), E)
    return (out3[0], out3[1], out3[2])
